# baseline (speedup 1.0000x reference)
import functools

import jax
import jax.numpy as jnp
from jax import lax
from jax.experimental import pallas as pl
from jax.experimental.pallas import tpu as pltpu
from jax.experimental.pallas import tpu_sc as plsc

_D = 64
_W = _D // 2
_S = 200
_CHUNKS = (104, 96)
_NCHUNK = len(_CHUNKS)
_NC = 2
_NS = 16
_NW = _NC * _NS
_ROWLOOK = 4
_LANES = 16
_PACK_CH = 125

_SC_PARAMS = pltpu.CompilerParams(
    use_tc_tiling_on_sc=False, needs_layout_passes=False
)


@jax.jit
def _cbow_sc(ids, table):
    V = table.shape[0]
    B = ids.shape[0] // _S
    R = B // _NW
    TV = V // _NW
    NCHK = TV // _PACK_CH

    mesh = plsc.VectorSubcoreMesh(core_axis_name="c", subcore_axis_name="s")

    @functools.partial(
        pl.kernel,
        out_type=jax.ShapeDtypeStruct((V, _W), jnp.int32),
        mesh=mesh,
        scratch_types=[pltpu.VMEM((2, _PACK_CH, _D), jnp.float32)]
        + [pltpu.VMEM((2, _PACK_CH, _W), jnp.int32)]
        + [pltpu.SemaphoreType.DMA for _ in range(4)],
        compiler_params=_SC_PARAMS,
    )
    def pack_table(table_hbm, packed_hbm, in_v, out_v, si0, si1, so0, so1):
        sis = (si0, si1)
        sos = (so0, so1)
        wid = lax.axis_index("s") * _NC + lax.axis_index("c")
        base = wid * TV

        half = jnp.int32(0x8000)
        hi_mask = jnp.int32(-65536)
        sixteen = jnp.int32(16)

        def fetch(chunk, b):
            pltpu.async_copy(
                table_hbm.at[pl.ds(base + chunk * _PACK_CH, _PACK_CH)],
                in_v.at[b],
                sis[b],
            )

        def pack_chunk(b):
            def body(j, _):
                for g in range(_D // 32):
                    s0 = in_v[b, j, pl.ds(32 * g, _LANES)]
                    s1 = in_v[b, j, pl.ds(32 * g + _LANES, _LANES)]
                    t0 = plsc.bitcast(s0, jnp.int32) + half
                    t1 = plsc.bitcast(s1, jnp.int32) + half
                    out_v[b, j, pl.ds(_LANES * g, _LANES)] = (
                        lax.shift_right_logical(t0, sixteen) | (t1 & hi_mask)
                    )
                return 0

            lax.fori_loop(0, _PACK_CH, body, 0)

        def put(chunk, b):
            pltpu.async_copy(
                out_v.at[b],
                packed_hbm.at[pl.ds(base + chunk * _PACK_CH, _PACK_CH)],
                sos[b],
            )

        def wait_fetch(b):
            pltpu.make_async_copy(
                table_hbm.at[pl.ds(0, _PACK_CH)], in_v.at[b], sis[b]
            ).wait()

        def wait_put(b):
            pltpu.make_async_copy(
                out_v.at[b], packed_hbm.at[pl.ds(0, _PACK_CH)], sos[b]
            ).wait()

        fetch(0, 0)

        def outer(i, _):
            for b in range(2):
                c = 2 * i + b

                @pl.when(c < NCHK)
                def _():
                    wait_fetch(b)

                    @pl.when(c + 1 < NCHK)
                    def _():
                        fetch(c + 1, 1 - b)

                    @pl.when(c >= 2)
                    def _():
                        wait_put(b)

                    pack_chunk(b)
                    put(c, b)

            return 0

        lax.fori_loop(0, (NCHK + 1) // 2, outer, 0)
        wait_put(0)

        @pl.when(NCHK > 1)
        def _():
            wait_put(1)

    @functools.partial(
        pl.kernel,
        out_type=jax.ShapeDtypeStruct((B, _D), jnp.float32),
        mesh=mesh,
        scratch_types=[
            pltpu.VMEM((R * _S,), jnp.int32),
            pltpu.VMEM((R, _D), jnp.float32),
        ]
        + [
            pltpu.VMEM((_CHUNKS[c], _W), jnp.int32)
            for _ in range(_ROWLOOK)
            for c in range(_NCHUNK)
        ]
        + [pltpu.SemaphoreType.DMA for _ in range(_ROWLOOK * _NCHUNK)],
        compiler_params=_SC_PARAMS,
    )
    def cbow(ids_hbm, table_hbm, out_hbm, idx_v, out_v, *rest):
        nstg = _ROWLOOK * _NCHUNK
        bufs = rest[:nstg]
        sems = rest[nstg:]
        wid = lax.axis_index("s") * _NC + lax.axis_index("c")
        base = wid * R

        pltpu.sync_copy(ids_hbm.at[pl.ds(base * _S, R * _S)], idx_v)

        def issue(row, c, p):
            off = row * _S + c * _CHUNKS[0]
            pltpu.async_copy(
                table_hbm.at[idx_v.at[pl.ds(off, _CHUNKS[c])]],
                bufs[p],
                sems[p],
            )

        def drain(c, p):
            pltpu.make_async_copy(
                table_hbm.at[idx_v.at[pl.ds(0, _CHUNKS[c])]],
                bufs[p],
                sems[p],
            ).wait()

        hi_mask = jnp.int32(-65536)
        sixteen = jnp.int32(16)

        def reduce_buf(buf, n, accs):
            def body(jj, accs):
                a0, a1, a2, a3 = accs
                for u in range(4):
                    j = jj * 4 + u
                    x0 = buf[j, pl.ds(0, _LANES)]
                    x1 = buf[j, pl.ds(_LANES, _LANES)]
                    a0 = a0 + plsc.bitcast(x0 << sixteen, jnp.float32)
                    a1 = a1 + plsc.bitcast(x0 & hi_mask, jnp.float32)
                    a2 = a2 + plsc.bitcast(x1 << sixteen, jnp.float32)
                    a3 = a3 + plsc.bitcast(x1 & hi_mask, jnp.float32)
                return (a0, a1, a2, a3)

            return lax.fori_loop(0, n // 4, body, accs)

        scale = jnp.float32(1.0 / _S)

        for k in range(_ROWLOOK):
            for c in range(_NCHUNK):
                issue(k, c, k * _NCHUNK + c)

        def outer(i, _):
            r0 = i * _ROWLOOK
            for k in range(_ROWLOOK):
                r = r0 + k
                z = jnp.zeros((_LANES,), jnp.float32)
                accs = (z, z, z, z)
                for c in range(_NCHUNK):
                    p = k * _NCHUNK + c
                    drain(c, p)
                    accs = reduce_buf(bufs[p], _CHUNKS[c], accs)

                    @pl.when(r + _ROWLOOK < R)
                    def _():
                        issue(r + _ROWLOOK, c, p)

                a0, a1, a2, a3 = accs
                out_v[r, pl.ds(0, _LANES)] = a0 * scale
                out_v[r, pl.ds(_LANES, _LANES)] = a1 * scale
                out_v[r, pl.ds(2 * _LANES, _LANES)] = a2 * scale
                out_v[r, pl.ds(3 * _LANES, _LANES)] = a3 * scale
            return 0

        lax.fori_loop(0, R // _ROWLOOK, outer, 0)

        pltpu.sync_copy(out_v, out_hbm.at[pl.ds(base, R)])

    return cbow(ids, pack_table(table))


def kernel(input_ids, table):
    B = input_ids.shape[0]
    return _cbow_sc(input_ids.astype(jnp.int32).reshape(B * _S), table)


# speedup vs baseline: 1.5052x; 1.5052x over previous
import functools

import numpy as np
import jax
import jax.numpy as jnp
from jax import lax
from jax.experimental import pallas as pl
from jax.experimental import layout as jex_layout
from jax.experimental.pallas import tpu as pltpu
from jax.experimental.pallas import tpu_sc as plsc

_D = 64
_S = 200
_CHUNKS = (104, 96)
_NCHUNK = len(_CHUNKS)
_NC = 2
_NS = 16
_NW = _NC * _NS
_ROWLOOK = 4
_LANES = 16

_PERM = np.empty((_D,), dtype=np.int32)
for _g in range(_D // 32):
    for _i in range(16):
        _PERM[32 * _g + _i] = 32 * _g + 2 * _i
        _PERM[32 * _g + 16 + _i] = 32 * _g + 2 * _i + 1
_INV = np.argsort(_PERM).astype(np.int32)

_ROW_MAJOR = jex_layout.Layout(major_to_minor=(0, 1))


@jax.jit
def _cbow_sc(ids, table_bf):
    B = ids.shape[0]
    R = B // _NW

    mesh = plsc.VectorSubcoreMesh(core_axis_name="c", subcore_axis_name="s")

    @functools.partial(
        pl.kernel,
        out_type=jax.ShapeDtypeStruct((B, _D), jnp.float32),
        mesh=mesh,
        scratch_types=[
            pltpu.VMEM((R, _S), jnp.int32),
            pltpu.VMEM((R, _D), jnp.float32),
        ]
        + [
            pltpu.VMEM((_CHUNKS[c], _D), jnp.bfloat16)
            for _ in range(_ROWLOOK)
            for c in range(_NCHUNK)
        ]
        + [pltpu.SemaphoreType.DMA for _ in range(_ROWLOOK * _NCHUNK)],
        compiler_params=pltpu.CompilerParams(
            use_tc_tiling_on_sc=False, needs_layout_passes=False
        ),
    )
    def cbow(ids_hbm, table_hbm, out_hbm, idx_v, out_v, *rest):
        nstg = _ROWLOOK * _NCHUNK
        bufs = rest[:nstg]
        sems = rest[nstg:]
        wid = lax.axis_index("s") * _NC + lax.axis_index("c")
        base = wid * R

        pltpu.sync_copy(ids_hbm.at[pl.ds(base, R)], idx_v)

        def issue(row, c, p):
            off = c * _CHUNKS[0]
            pltpu.async_copy(
                table_hbm.at[idx_v.at[row, pl.ds(off, _CHUNKS[c])]],
                bufs[p],
                sems[p],
            )

        def drain(c, p):
            pltpu.make_async_copy(
                table_hbm.at[idx_v.at[0, pl.ds(0, _CHUNKS[c])]],
                bufs[p],
                sems[p],
            ).wait()

        hi_mask = jnp.int32(-65536)
        sixteen = jnp.int32(16)

        def reduce_buf(buf, n, accs):
            def body(jj, accs):
                a0, a1, a2, a3 = accs
                for u in range(4):
                    j = jj * 4 + u
                    x0 = plsc.bitcast(buf[j, pl.ds(0, 32)], jnp.int32)
                    x1 = plsc.bitcast(buf[j, pl.ds(32, 32)], jnp.int32)
                    a0 = a0 + plsc.bitcast(x0 << sixteen, jnp.float32)
                    a1 = a1 + plsc.bitcast(x0 & hi_mask, jnp.float32)
                    a2 = a2 + plsc.bitcast(x1 << sixteen, jnp.float32)
                    a3 = a3 + plsc.bitcast(x1 & hi_mask, jnp.float32)
                return (a0, a1, a2, a3)

            return lax.fori_loop(0, n // 4, body, accs)

        scale = jnp.float32(1.0 / _S)

        for k in range(_ROWLOOK):
            for c in range(_NCHUNK):
                issue(k, c, k * _NCHUNK + c)

        def outer(i, _):
            r0 = i * _ROWLOOK
            for k in range(_ROWLOOK):
                r = r0 + k
                z = jnp.zeros((_LANES,), jnp.float32)
                accs = (z, z, z, z)
                for c in range(_NCHUNK):
                    p = k * _NCHUNK + c
                    drain(c, p)
                    accs = reduce_buf(bufs[p], _CHUNKS[c], accs)

                    @pl.when(r + _ROWLOOK < R)
                    def _():
                        issue(r + _ROWLOOK, c, p)

                a0, a1, a2, a3 = accs
                out_v[r, pl.ds(0, _LANES)] = a0 * scale
                out_v[r, pl.ds(_LANES, _LANES)] = a1 * scale
                out_v[r, pl.ds(2 * _LANES, _LANES)] = a2 * scale
                out_v[r, pl.ds(3 * _LANES, _LANES)] = a3 * scale
            return 0

        lax.fori_loop(0, R // _ROWLOOK, outer, 0)

        pltpu.sync_copy(out_v, out_hbm.at[pl.ds(base, R)])

    return cbow(ids, table_bf)


def kernel(input_ids, table):
    table_bf = jex_layout.with_layout_constraint(
        table.astype(jnp.bfloat16), _ROW_MAJOR
    )
    out = _cbow_sc(input_ids.astype(jnp.int32), table_bf)
    return out[:, _INV]
